# baseline (speedup 1.0000x reference)
import jax
import jax.numpy as jnp
from jax.experimental import pallas as pl
from jax.experimental.pallas import tpu as pltpu

BM = 2048
NSTRIPE = 4
NSLOT = 2


def _copy_in(x_hbm, xbuf, sems, tile, slot):
    rows = BM // NSTRIPE
    for s in range(NSTRIPE):
        pltpu.make_async_copy(
            x_hbm.at[pl.ds(tile * BM + s * rows, rows), :],
            xbuf.at[slot, pl.ds(s * rows, rows), :],
            sems.at[slot, s],
        ).start()


def _wait_in(x_hbm, xbuf, sems, tile, slot):
    rows = BM // NSTRIPE
    for s in range(NSTRIPE):
        pltpu.make_async_copy(
            x_hbm.at[pl.ds(tile * BM + s * rows, rows), :],
            xbuf.at[slot, pl.ds(s * rows, rows), :],
            sems.at[slot, s],
        ).wait()


def _copy_out(o_hbm, obuf, sems, tile, slot):
    rows = BM // NSTRIPE
    for s in range(NSTRIPE):
        pltpu.make_async_copy(
            obuf.at[slot, pl.ds(s * rows, rows), :],
            o_hbm.at[pl.ds(tile * BM + s * rows, rows), :],
            sems.at[slot, s],
        ).start()


def _wait_out(o_hbm, obuf, sems, tile, slot):
    rows = BM // NSTRIPE
    for s in range(NSTRIPE):
        pltpu.make_async_copy(
            obuf.at[slot, pl.ds(s * rows, rows), :],
            o_hbm.at[pl.ds(tile * BM + s * rows, rows), :],
            sems.at[slot, s],
        ).wait()


def _mlp_tile(x, w1, b1, w2, b2, w3, b3, w4, b4):
    h = jnp.dot(x.astype(jnp.bfloat16), w1,
                preferred_element_type=jnp.float32) + b1
    h = jnp.maximum(h, 0.0)
    h = jnp.dot(h.astype(jnp.bfloat16), w2,
                preferred_element_type=jnp.float32) + b2
    h = h * jax.nn.sigmoid(h)
    h = jnp.dot(h.astype(jnp.bfloat16), w3,
                preferred_element_type=jnp.float32) + b3
    h = h * jax.nn.sigmoid(h)
    h = jnp.dot(h.astype(jnp.bfloat16), w4,
                preferred_element_type=jnp.float32) + b4
    return h


def _body(x_hbm, w1_ref, b1_ref, w2_ref, b2_ref, w3_ref, b3_ref,
          w4_ref, b4_ref, o_hbm, xbuf, obuf, in_sems, out_sems):
    n_tiles = x_hbm.shape[0] // BM
    w1, b1 = w1_ref[...], b1_ref[...]
    w2, b2 = w2_ref[...], b2_ref[...]
    w3, b3 = w3_ref[...], b3_ref[...]
    w4, b4 = w4_ref[...], b4_ref[...]

    _copy_in(x_hbm, xbuf, in_sems, 0, 0)
    for i in range(n_tiles):
        slot = i % NSLOT
        if i + 1 < n_tiles:
            _copy_in(x_hbm, xbuf, in_sems, i + 1, (i + 1) % NSLOT)
        _wait_in(x_hbm, xbuf, in_sems, i, slot)
        if i >= NSLOT:
            _wait_out(o_hbm, obuf, out_sems, i - NSLOT, slot)
        obuf[slot] = _mlp_tile(xbuf[slot], w1, b1, w2, b2, w3, b3, w4, b4)
        _copy_out(o_hbm, obuf, out_sems, i, slot)
    for i in range(max(0, n_tiles - NSLOT), n_tiles):
        _wait_out(o_hbm, obuf, out_sems, i, i % NSLOT)


def kernel(t, x_flat, W1, b1, W2, b2, W3, b3, W4, b4):
    del t
    B, D = x_flat.shape
    H = W1.shape[1]

    vm = pl.BlockSpec(memory_space=pltpu.MemorySpace.VMEM)
    anym = pl.BlockSpec(memory_space=pltpu.MemorySpace.HBM)

    return pl.pallas_call(
        _body,
        in_specs=[anym, vm, vm, vm, vm, vm, vm, vm, vm],
        out_specs=anym,
        out_shape=jax.ShapeDtypeStruct((B, D), jnp.float32),
        scratch_shapes=[
            pltpu.VMEM((NSLOT, BM, D), jnp.float32),
            pltpu.VMEM((NSLOT, BM, D), jnp.float32),
            pltpu.SemaphoreType.DMA((NSLOT, NSTRIPE)),
            pltpu.SemaphoreType.DMA((NSLOT, NSTRIPE)),
        ],
    )(x_flat,
      W1.astype(jnp.bfloat16), b1.reshape(1, H),
      W2.astype(jnp.bfloat16), b2.reshape(1, H),
      W3.astype(jnp.bfloat16), b3.reshape(1, H),
      W4.astype(jnp.bfloat16), b4.reshape(1, D))
